# baseline (speedup 1.0000x reference)
import functools

import jax
import jax.numpy as jnp
from jax import lax
from jax.experimental import pallas as pl
from jax.experimental.pallas import tpu as pltpu
from jax.experimental.pallas import tpu_sc as plsc

INPUT_DIM = 4096
NUM_EXPERTS = 64
K = 8
BLK = 1024
NUM_CORES = 2
NUM_SUBCORES = 16
LANES = 16
NUM_WORKERS = NUM_CORES * NUM_SUBCORES
SC_CHUNK = 512


def _tc_body(x_ref, w_ref, b_ref, o_ref):
    logits = lax.dot_general(
        w_ref[...], x_ref[...],
        (((1,), (1,)), ((), ())),
        preferred_element_type=jnp.float32,
    ) + b_ref[...]
    m = jnp.max(logits, axis=0, keepdims=True)
    e = jnp.exp(logits - m)
    o_ref[...] = e / jnp.sum(e, axis=0, keepdims=True)


def _sc_body(rwT_hbm, out_hbm, rw_v, out_v, sem):
    wid = lax.axis_index("s") * NUM_CORES + lax.axis_index("c")
    tokens_per_worker = rwT_hbm.shape[1] // NUM_WORKERS

    for chunk in range(tokens_per_worker // SC_CHUNK):
        base = wid * tokens_per_worker + chunk * SC_CHUNK
        pltpu.async_copy(
            rwT_hbm.at[:, pl.ds(base, SC_CHUNK)], rw_v, sem).wait()

        def group_body(g, _):
            off = g * LANES
            vals = [rw_v[e, pl.ds(off, LANES)] for e in range(NUM_EXPERTS)]
            cur = vals
            thresh = None
            for _i in range(K):
                thresh = cur[0]
                for e in range(1, NUM_EXPERTS):
                    thresh = jnp.maximum(thresh, cur[e])
                if _i < K - 1:
                    cur = [jnp.where(v >= thresh, -1.0, v) for v in cur]
            rows = (off + lax.iota(jnp.int32, LANES)) * NUM_EXPERTS
            for e in range(NUM_EXPERTS):
                masked = jnp.where(vals[e] >= thresh, vals[e], 0.0)
                plsc.store_scatter(out_v, [rows + e], masked)
            return 0

        lax.fori_loop(0, SC_CHUNK // LANES, group_body, 0)
        pltpu.sync_copy(
            out_v,
            out_hbm.at[pl.ds(base * NUM_EXPERTS, SC_CHUNK * NUM_EXPERTS)])


@jax.jit
def kernel(x, W, b):
    B, S, D = x.shape
    E = W.shape[0]
    N = B * S
    x2 = x.reshape(N, D)

    rwT = pl.pallas_call(
        _tc_body,
        grid=(N // BLK,),
        in_specs=[
            pl.BlockSpec((BLK, D), lambda i: (i, 0)),
            pl.BlockSpec((E, D), lambda i: (0, 0)),
            pl.BlockSpec((E, 1), lambda i: (0, 0)),
        ],
        out_specs=pl.BlockSpec((E, BLK), lambda i: (0, i)),
        out_shape=jax.ShapeDtypeStruct((E, N), jnp.float32),
    )(x2, W, b.reshape(E, 1))

    mesh = plsc.VectorSubcoreMesh(
        core_axis_name="c", subcore_axis_name="s",
        num_cores=NUM_CORES, num_subcores=NUM_SUBCORES)
    sc_topk = functools.partial(
        pl.kernel,
        out_type=jax.ShapeDtypeStruct((N * E,), jnp.float32),
        mesh=mesh,
        scratch_types=[
            pltpu.VMEM((E, SC_CHUNK), jnp.float32),
            pltpu.VMEM((SC_CHUNK * E,), jnp.float32),
            pltpu.SemaphoreType.DMA,
        ],
        compiler_params=pltpu.CompilerParams(needs_layout_passes=False),
    )(_sc_body)
    out = sc_topk(rwT)
    return out.reshape(B, S, E)
